# baseline (speedup 1.0000x reference)
import functools

import jax
import jax.numpy as jnp
from jax.experimental import pallas as pl
from jax.experimental.pallas import tpu as pltpu

_RADIUS = 16
_STRIDE = 2
_EPS = 1e-5


def _kernel_body(f_ref, p8_ref, pc_ref,
                 w1a_ref, w1b_ref, g1_ref, b1_ref, wa_ref,
                 w2_ref, g2_ref, b2_ref, out_ref,
                 ae_s, ao_s, ye_s, yo_s, af_s,
                 *, B, P, H, C):
    Pp = P + 2 * H
    n_bn2 = B * P * 2
    n_bn1 = B * P * 2 * _RADIUS

    w1a = w1a_ref[...]
    w1b = w1b_ref[...]
    w2 = w2_ref[...]

    feats = f_ref[...]
    pts8 = p8_ref[...]

    def deinterleave(dst_e, dst_o):
        for dst, off in ((dst_e, 0), (dst_o, 1)):
            core = af_s[off::2, :].reshape(B, P, C)
            dst[:, H:H + P, :] = core
            dst[:, 0:H, :] = core[:, P - H:P, :]
            dst[:, H + P:Pp, :] = core[:, 0:H, :]

    af_s[...] = (jnp.dot(feats, w1b, preferred_element_type=jnp.float32)
                 + jnp.dot(pts8, w1a, preferred_element_type=jnp.float32))
    deinterleave(ae_s, ao_s)

    pw_c = jnp.dot(pc_ref[...].reshape(B * P, 8), w1a,
                   preferred_element_type=jnp.float32).reshape(B, P, C)

    ta_s = ye_s
    tb_s = yo_s

    def window16(src):
        ta_s[:, 0:Pp - 1, :] = src[:, 0:Pp - 1, :] + src[:, 1:Pp, :]
        tb_s[:, 0:Pp - 3, :] = ta_s[:, 0:Pp - 3, :] + ta_s[:, 2:Pp - 1, :]
        ta_s[:, 0:Pp - 7, :] = tb_s[:, 0:Pp - 7, :] + tb_s[:, 4:Pp - 3, :]
        return ta_s[:, 0:P + 1, :] + ta_s[:, 8:P + 9, :]

    t8_o = window16(ao_s)[:, 0:P, :]
    t8_e = window16(ae_s)[:, 0:P, :]
    u = (t8_e + ae_s[:, 16:16 + P, :] - ae_s[:, 8:8 + P, :] + t8_o)

    a_v = af_s[...]
    sum_a = jnp.sum(a_v, axis=0, keepdims=True)
    sum_a2 = jnp.sum(a_v * a_v, axis=0, keepdims=True)
    pw2 = pw_c.reshape(B * P, C)
    sum_pw = jnp.sum(pw2, axis=0, keepdims=True)
    sum_pw2 = jnp.sum(pw2 * pw2, axis=0, keepdims=True)
    cross = jnp.sum(pw2 * u.reshape(B * P, C), axis=0, keepdims=True)

    S = 16.0 * sum_a - 32.0 * sum_pw
    Q = 16.0 * sum_a2 - 2.0 * cross + 32.0 * sum_pw2
    mu1 = S / n_bn1
    var1 = Q / n_bn1 - mu1 * mu1
    scale1 = g1_ref[...] * jax.lax.rsqrt(var1 + _EPS)
    shift1 = b1_ref[...] - mu1 * scale1

    ae_s[...] = ae_s[...] * scale1
    ao_s[...] = ao_s[...] * scale1
    pwn = pw_c * scale1 - shift1

    Z = jnp.dot(feats, w2, preferred_element_type=jnp.float32)
    s2 = jnp.sum(Z, axis=0, keepdims=True)
    q2 = jnp.sum(Z * Z, axis=0, keepdims=True)
    mu2 = s2 / n_bn2
    var2 = q2 / n_bn2 - mu2 * mu2
    scale2 = g2_ref[...] * jax.lax.rsqrt(var2 + _EPS)
    shift2 = b2_ref[...] - mu2 * scale2
    af_s[...] = jnp.maximum(Z * scale2 + shift2, 0.0)
    deinterleave(ye_s, yo_s)

    wa_col = wa_ref[...]

    def logit(aw, o):
        hn = jnp.maximum(aw[:, o:o + P, :] - pwn, 0.0)
        return jnp.dot(hn.reshape(B * P, C), wa_col,
                       preferred_element_type=jnp.float32).reshape(B, P, 1)

    def pair_contrib(aw, yw, o1, o2):
        l12 = jnp.concatenate([logit(aw, o1), logit(aw, o2)], axis=2)
        e12 = jnp.exp(l12)
        e1 = e12[:, :, 0:1]
        e2 = e12[:, :, 1:2]
        return e1 + e2, e1 * yw[:, o1:o1 + P, :] + e2 * yw[:, o2:o2 + P, :]

    def pair_body(src, ysrc, base, i, carry):
        denom, acc = carry
        aw = src[:, pl.ds(base + i, P + 8), :]
        yw = ysrc[:, pl.ds(base + i, P + 8), :]
        de, dc = pair_contrib(aw, yw, 0, 8)
        return (denom + de, acc + dc)

    denom = jnp.zeros((B, P, 1), jnp.float32)
    acc = jnp.zeros((B, P, C), jnp.float32)
    denom, acc = jax.lax.fori_loop(
        0, 7, functools.partial(pair_body, ae_s, ye_s, 1), (denom, acc))
    de, dc = pair_contrib(ae_s[...], ye_s[...], 0, 16)
    denom = denom + de
    acc = acc + dc
    denom, acc = jax.lax.fori_loop(
        0, 8, functools.partial(pair_body, ao_s, yo_s, 0), (denom, acc))

    out_ref[...] = acc / denom


def kernel(points, features, W1, g1, b1, Wa, ba, W2, g2, b2):
    B, N, _ = points.shape
    C = features.shape[1]
    P = N // _STRIDE
    H = _RADIUS // 2

    pts8 = jnp.pad(points.reshape(B * N, 2), ((0, 0), (0, 6)))
    pc8 = jnp.pad(points[:, ::_STRIDE], ((0, 0), (0, 0), (0, 6)))

    w1a = jnp.pad(W1[:2], ((0, 6), (0, 0)))
    w1b = W1[2:]

    Pp = P + 2 * H
    out = pl.pallas_call(
        functools.partial(_kernel_body, B=B, P=P, H=H, C=C),
        out_shape=jax.ShapeDtypeStruct((B, P, C), jnp.float32),
        scratch_shapes=[pltpu.VMEM((B, Pp, C), jnp.float32)] * 4
        + [pltpu.VMEM((B * N, C), jnp.float32)],
        compiler_params=pltpu.CompilerParams(vmem_limit_bytes=66_900_000),
    )(features, pts8, pc8,
      w1a, w1b, g1.reshape(1, C), b1.reshape(1, C), Wa,
      W2, g2.reshape(1, C), b2.reshape(1, C))

    pts_out = points[:, ::_STRIDE]
    return (pts_out, out.reshape(B * P, C))


# speedup vs baseline: 2.2250x; 2.2250x over previous
import functools

import jax
import jax.numpy as jnp
from jax.experimental import pallas as pl
from jax.experimental.pallas import tpu as pltpu

_RADIUS = 16
_STRIDE = 2
_EPS = 1e-5


def _kernel_body(f_ref, p8_ref, pc_ref,
                 w1a_ref, w1b_ref, g1_ref, b1_ref, wa_ref,
                 w2_ref, g2_ref, b2_ref, out_ref,
                 ae_s, ao_s, ye_s, yo_s, af_s,
                 *, B, P, H, C):
    Pp = P + 2 * H
    n_bn2 = B * P * 2
    n_bn1 = B * P * 2 * _RADIUS

    w1a = w1a_ref[...]
    w1b = w1b_ref[...]
    w2 = w2_ref[...]

    feats = f_ref[...]
    pts8 = p8_ref[...]

    def deinterleave(dst_e, dst_o):
        for dst, off in ((dst_e, 0), (dst_o, 1)):
            core = af_s[off::2, :].reshape(B, P, C)
            dst[:, H:H + P, :] = core
            dst[:, 0:H, :] = core[:, P - H:P, :]
            dst[:, H + P:Pp, :] = core[:, 0:H, :]

    af_s[...] = (jnp.dot(feats, w1b, preferred_element_type=jnp.float32)
                 + jnp.dot(pts8, w1a, preferred_element_type=jnp.float32))
    deinterleave(ae_s, ao_s)

    pw_c = jnp.dot(pc_ref[...].reshape(B * P, 8), w1a,
                   preferred_element_type=jnp.float32).reshape(B, P, C)

    ta_s = ye_s
    tb_s = yo_s

    def window16(src):
        ta_s[:, 0:Pp - 1, :] = src[:, 0:Pp - 1, :] + src[:, 1:Pp, :]
        tb_s[:, 0:Pp - 3, :] = ta_s[:, 0:Pp - 3, :] + ta_s[:, 2:Pp - 1, :]
        ta_s[:, 0:Pp - 7, :] = tb_s[:, 0:Pp - 7, :] + tb_s[:, 4:Pp - 3, :]
        return ta_s[:, 0:P + 1, :] + ta_s[:, 8:P + 9, :]

    t8_o = window16(ao_s)[:, 0:P, :]
    t8_e = window16(ae_s)[:, 0:P, :]
    u = (t8_e + ae_s[:, 16:16 + P, :] - ae_s[:, 8:8 + P, :] + t8_o)

    a_v = af_s[...]
    sum_a = jnp.sum(a_v, axis=0, keepdims=True)
    sum_a2 = jnp.sum(a_v * a_v, axis=0, keepdims=True)
    pw2 = pw_c.reshape(B * P, C)
    sum_pw = jnp.sum(pw2, axis=0, keepdims=True)
    sum_pw2 = jnp.sum(pw2 * pw2, axis=0, keepdims=True)
    cross = jnp.sum(pw2 * u.reshape(B * P, C), axis=0, keepdims=True)

    S = 16.0 * sum_a - 32.0 * sum_pw
    Q = 16.0 * sum_a2 - 2.0 * cross + 32.0 * sum_pw2
    mu1 = S / n_bn1
    var1 = Q / n_bn1 - mu1 * mu1
    scale1 = g1_ref[...] * jax.lax.rsqrt(var1 + _EPS)
    shift1 = b1_ref[...] - mu1 * scale1

    ae_s[...] = ae_s[...] * scale1
    ao_s[...] = ao_s[...] * scale1
    pwn = pw_c * scale1 - shift1

    Z = jnp.dot(feats, w2, preferred_element_type=jnp.float32)
    s2 = jnp.sum(Z, axis=0, keepdims=True)
    q2 = jnp.sum(Z * Z, axis=0, keepdims=True)
    mu2 = s2 / n_bn2
    var2 = q2 / n_bn2 - mu2 * mu2
    scale2 = g2_ref[...] * jax.lax.rsqrt(var2 + _EPS)
    shift2 = b2_ref[...] - mu2 * scale2
    af_s[...] = jnp.maximum(Z * scale2 + shift2, 0.0)
    deinterleave(ye_s, yo_s)

    wa = wa_ref[...]

    def contrib(aw, yw, o):
        hn = jnp.maximum(aw[:, o:o + P, :] - pwn, 0.0)
        e = jnp.exp(jnp.sum(hn * wa, axis=2, keepdims=True))
        return e, e * yw[:, o:o + P, :]

    denom = jnp.zeros((B, P, 1), jnp.float32)
    acc = jnp.zeros((B, P, C), jnp.float32)
    for src, ysrc, starts in ((ae_s, ye_s, tuple(t for t in range(17) if t != 8)),
                              (ao_s, yo_s, tuple(range(16)))):
        for t in starts:
            e, c = contrib(src, ysrc, t)
            denom = denom + e
            acc = acc + c

    out_ref[...] = acc / denom


def kernel(points, features, W1, g1, b1, Wa, ba, W2, g2, b2):
    B, N, _ = points.shape
    C = features.shape[1]
    P = N // _STRIDE
    H = _RADIUS // 2

    pts8 = jnp.pad(points.reshape(B * N, 2), ((0, 0), (0, 6)))
    pc8 = jnp.pad(points[:, ::_STRIDE], ((0, 0), (0, 0), (0, 6)))

    w1a = jnp.pad(W1[:2], ((0, 6), (0, 0)))
    w1b = W1[2:]

    Pp = P + 2 * H
    out = pl.pallas_call(
        functools.partial(_kernel_body, B=B, P=P, H=H, C=C),
        out_shape=jax.ShapeDtypeStruct((B, P, C), jnp.float32),
        scratch_shapes=[pltpu.VMEM((B, Pp, C), jnp.float32)] * 4
        + [pltpu.VMEM((B * N, C), jnp.float32)],
        compiler_params=pltpu.CompilerParams(vmem_limit_bytes=66_900_000),
    )(features, pts8, pc8,
      w1a, w1b, g1.reshape(1, C), b1.reshape(1, C), Wa.reshape(1, C),
      W2, g2.reshape(1, C), b2.reshape(1, C))

    pts_out = points[:, ::_STRIDE]
    return (pts_out, out.reshape(B * P, C))
